# baseline (speedup 1.0000x reference)
import functools

import jax
import jax.numpy as jnp
from jax import lax
from jax.experimental import pallas as pl
from jax.experimental.pallas import tpu as pltpu
from jax.experimental.pallas import tpu_sc as plsc

B = 16384
L = 200
LP = 208
E = 64
NC = 2
NS = 16
NW = NC * NS
RPW = B // NW
D1, D2 = 256, 128
BT = 1024


def _sc_features(xseq, i_uid, i_gen, i_city, i_iid, i_cate,
                 t_uid, t_gen, t_city, t_iid, t_cate,
                 o_uid, o_gen, o_city, o_iid, o_cate, o_seq,
                 idxf_v, rows_v, sidx_v, sbuf_v, e0_v, semf, sem0, sem1):
  wid = lax.axis_index("s") * NC + lax.axis_index("c")
  base = wid * RPW

  for idx_hbm, tab, out in ((i_uid, t_uid, o_uid), (i_gen, t_gen, o_gen),
                            (i_city, t_city, o_city), (i_iid, t_iid, o_iid),
                            (i_cate, t_cate, o_cate)):
    pltpu.sync_copy(idx_hbm.at[pl.ds(base, RPW)], idxf_v)
    for c in range(RPW // 128):
      pltpu.make_async_copy(tab.at[idxf_v.at[pl.ds(c * 128, 128)]],
                            rows_v.at[pl.ds(c * 128, 128)], semf).start()
    for c in range(RPW // 128):
      pltpu.make_async_copy(tab.at[idxf_v.at[pl.ds(c * 128, 128)]],
                            rows_v.at[pl.ds(c * 128, 128)], semf).wait()
    pltpu.sync_copy(rows_v, out.at[pl.ds(base, RPW)])

  pltpu.sync_copy(t_iid.at[0], e0_v)
  sems = (sem0, sem1)

  def _gathers(s):
    return ((t_iid.at[sidx_v.at[s, pl.ds(0, 128)]],
             sbuf_v.at[s, pl.ds(0, 128)], sems[s]),
            (t_iid.at[sidx_v.at[s, pl.ds(128, LP - 128)]],
             sbuf_v.at[s, pl.ds(128, LP - 128)], sems[s]))

  def issue(r, s):
    pltpu.sync_copy(xseq.at[base + r], sidx_v.at[s])
    for src, dst, sem in _gathers(s):
      pltpu.make_async_copy(src, dst, sem).start()

  def drain(s):
    for src, dst, sem in _gathers(s):
      pltpu.make_async_copy(src, dst, sem).wait()

  def compute(r, s):
    nz = jnp.zeros((16,), jnp.float32)
    for c in range(LP // 16):
      ch = sidx_v[s, pl.ds(c * 16, 16)]
      nz = nz + jnp.where(ch == 0, jnp.float32(1.0), jnp.float32(0.0))
    lanes = lax.iota(jnp.int32, 16)
    dn = lax.GatherDimensionNumbers(offset_dims=(), collapsed_slice_dims=(0,),
                                    start_index_map=(0,))
    for shift in (8, 4, 2, 1):
      perm = jnp.bitwise_xor(lanes, shift)
      nz = nz + lax.gather(nz, perm[:, None], dn, (1,),
                           mode=lax.GatherScatterMode.PROMISE_IN_BOUNDS)
    n0v = nz
    rcp = jnp.float32(1.0) / (jnp.float32(LP) - n0v + jnp.float32(1e-8))

    def sbody(j, accs):
      out = list(accs)
      for u in range(8):
        row = j * 8 + u
        for c in range(4):
          out[c] = out[c] + sbuf_v[s, row, pl.ds(c * 16, 16)]
      return tuple(out)

    accs = lax.fori_loop(0, LP // 8, sbody,
                         tuple(jnp.zeros((16,), jnp.float32) for _ in range(4)))
    for c in range(4):
      avg = (accs[c] - n0v * e0_v[pl.ds(c * 16, 16)]) * rcp
      rows_v[r, pl.ds(c * 16, 16)] = avg

  issue(0, 0)
  issue(1, 1)

  def pair(i, carry):
    r = 2 * i
    drain(0)
    compute(r, 0)

    @pl.when(r + 2 < RPW)
    def _():
      issue(r + 2, 0)

    drain(1)
    compute(r + 1, 1)

    @pl.when(r + 3 < RPW)
    def _():
      issue(r + 3, 1)

    return carry

  lax.fori_loop(0, RPW // 2, pair, jnp.int32(0))
  pltpu.sync_copy(rows_v, o_seq.at[pl.ds(base, RPW)])


_sc_embed = functools.partial(
    pl.kernel,
    out_type=[jax.ShapeDtypeStruct((B, E), jnp.float32)] * 6,
    mesh=plsc.VectorSubcoreMesh(core_axis_name="c", subcore_axis_name="s"),
    compiler_params=pltpu.CompilerParams(use_tc_tiling_on_sc=False),
    scratch_types=[
        pltpu.VMEM((RPW,), jnp.int32),
        pltpu.VMEM((RPW, E), jnp.float32),
        pltpu.VMEM((2, LP), jnp.int32),
        pltpu.VMEM((2, LP, E), jnp.float32),
        pltpu.VMEM((E,), jnp.float32),
        pltpu.SemaphoreType.DMA,
        pltpu.SemaphoreType.DMA,
        pltpu.SemaphoreType.DMA,
    ],
)(_sc_features)


def _l1_body(f0, f1, f2, f3, f4, f5, xsc, w, wsc, b, h_ref, s_ref, q_ref):
  hid = jnp.concatenate(
      [f0[...], f1[...], f2[...], f3[...], f4[...], f5[...]], axis=1)
  h = jnp.dot(hid, w[...], preferred_element_type=jnp.float32)
  xv = xsc[...]
  wv = wsc[...]
  h = h + xv[:, 0:1] * wv[0:1, :] + xv[:, 1:2] * wv[1:2, :] + b[...]
  h_ref[...] = h

  @pl.when(pl.program_id(0) == 0)
  def _():
    s_ref[...] = jnp.zeros_like(s_ref)
    q_ref[...] = jnp.zeros_like(q_ref)

  s_ref[...] += jnp.sum(h, axis=0, keepdims=True)
  q_ref[...] += jnp.sum(h * h, axis=0, keepdims=True)


def _l2_body(h0, sc, sh, w, b, h_ref, s_ref, q_ref):
  a = jnp.maximum(h0[...] * sc[...] + sh[...], 0.0)
  h = jnp.dot(a, w[...], preferred_element_type=jnp.float32) + b[...]
  h_ref[...] = h

  @pl.when(pl.program_id(0) == 0)
  def _():
    s_ref[...] = jnp.zeros_like(s_ref)
    q_ref[...] = jnp.zeros_like(q_ref)

  s_ref[...] += jnp.sum(h, axis=0, keepdims=True)
  q_ref[...] += jnp.sum(h * h, axis=0, keepdims=True)


def _l3_body(h1, sc, sh, w, b, o_ref):
  a = jnp.maximum(h1[...] * sc[...] + sh[...], 0.0)
  o_ref[...] = jnp.dot(a, w[...], preferred_element_type=jnp.float32) + b[...]


def _full(shape):
  return pl.BlockSpec(shape, lambda i: (0, 0))


def _tile(width):
  return pl.BlockSpec((BT, width), lambda i: (i, 0))


def _layer1(feats, xsc, w, wsc, b):
  return pl.pallas_call(
      _l1_body,
      grid=(B // BT,),
      in_specs=[_tile(E)] * 6 + [_tile(2), _full((6 * E, 2 * D1)),
                                 _full((2, 2 * D1)), _full((1, 2 * D1))],
      out_specs=[_tile(2 * D1), _full((1, 2 * D1)), _full((1, 2 * D1))],
      out_shape=[jax.ShapeDtypeStruct((B, 2 * D1), jnp.float32),
                 jax.ShapeDtypeStruct((1, 2 * D1), jnp.float32),
                 jax.ShapeDtypeStruct((1, 2 * D1), jnp.float32)],
  )(*feats, xsc, w, wsc, b)


def _layer2(h0, sc, sh, w, b):
  return pl.pallas_call(
      _l2_body,
      grid=(B // BT,),
      in_specs=[_tile(2 * D1), _full((1, 2 * D1)), _full((1, 2 * D1)),
                _full((2 * D1, 2 * D2)), _full((1, 2 * D2))],
      out_specs=[_tile(2 * D2), _full((1, 2 * D2)), _full((1, 2 * D2))],
      out_shape=[jax.ShapeDtypeStruct((B, 2 * D2), jnp.float32),
                 jax.ShapeDtypeStruct((1, 2 * D2), jnp.float32),
                 jax.ShapeDtypeStruct((1, 2 * D2), jnp.float32)],
  )(h0, sc, sh, w, b)


def _layer3(h1, sc, sh, w, b):
  return pl.pallas_call(
      _l3_body,
      grid=(B // BT,),
      in_specs=[_tile(2 * D2), _full((1, 2 * D2)), _full((1, 2 * D2)),
                _full((2 * D2, 2)), _full((1, 2))],
      out_specs=_tile(2),
      out_shape=jax.ShapeDtypeStruct((B, 2), jnp.float32),
  )(h1, sc, sh, w, b)


def _bn_fold(s, q, g, be):
  mu = s / B
  var = q / B - mu * mu
  scale = g / jnp.sqrt(var + 1e-5)
  return scale, be - mu * scale


def kernel(x, x_seq, emb_user_id, emb_user_gender, emb_user_city, emb_item_id,
           emb_item_cate,
           t0_W0, t0_b0, t0_g0, t0_be0, t0_W1, t0_b1, t0_g1, t0_be1,
           t0_Wout, t0_bout,
           t1_W0, t1_b0, t1_g0, t1_be0, t1_W1, t1_b1, t1_g1, t1_be1,
           t1_Wout, t1_bout):
  xi = x.astype(jnp.int32)
  xseq_p = jnp.pad(x_seq.astype(jnp.int32), ((0, 0), (0, LP - L)))
  feats = _sc_embed(xseq_p, xi[:, 0], xi[:, 2], xi[:, 3], xi[:, 4], xi[:, 5],
                    emb_user_id, emb_user_gender, emb_user_city, emb_item_id,
                    emb_item_cate)
  xsc = jnp.stack([x[:, 1], x[:, 6]], axis=1)

  def _perm(W):
    We = jnp.concatenate([W[0:64], W[65:129], W[129:193], W[193:257],
                          W[257:321], W[322:386]], axis=0)
    return We, jnp.stack([W[64], W[321]], axis=0)

  W0e0, Wsc0 = _perm(t0_W0)
  W0e1, Wsc1 = _perm(t1_W0)
  W0cat = jnp.concatenate([W0e0, W0e1], axis=1)
  Wsccat = jnp.concatenate([Wsc0, Wsc1], axis=1)
  b0cat = jnp.concatenate([t0_b0, t1_b0])[None, :]
  g0cat = jnp.concatenate([t0_g0, t1_g0])[None, :]
  be0cat = jnp.concatenate([t0_be0, t1_be0])[None, :]
  W1bd = (jnp.zeros((2 * D1, 2 * D2), jnp.float32)
          .at[:D1, :D2].set(t0_W1).at[D1:, D2:].set(t1_W1))
  b1cat = jnp.concatenate([t0_b1, t1_b1])[None, :]
  g1cat = jnp.concatenate([t0_g1, t1_g1])[None, :]
  be1cat = jnp.concatenate([t0_be1, t1_be1])[None, :]
  Woutbd = (jnp.zeros((2 * D2, 2), jnp.float32)
            .at[:D2, 0:1].set(t0_Wout).at[D2:, 1:2].set(t1_Wout))
  boutcat = jnp.concatenate([t0_bout, t1_bout])[None, :]

  h0, s0, q0 = _layer1(feats, xsc, W0cat, Wsccat, b0cat)
  sc0, sh0 = _bn_fold(s0, q0, g0cat, be0cat)
  h1, s1, q1 = _layer2(h0, sc0, sh0, W1bd, b1cat)
  sc1, sh1 = _bn_fold(s1, q1, g1cat, be1cat)
  out = _layer3(h1, sc1, sh1, Woutbd, boutcat)
  return (out[:, 0:1], out[:, 1:2])
